# baseline (speedup 1.0000x reference)
import functools

import jax
import jax.numpy as jnp
from jax import lax
from jax.experimental import pallas as pl
from jax.experimental.pallas import tpu as pltpu
from jax.experimental.pallas import tpu_sc as plsc

_VOCAB = 100000
_CTX = 2048
_DIM = 1024
_BATCH = 4

_NC = 2
_NS = 16
_NW = _NC * _NS
_PW = _CTX // _NW
_SUB = 32
_HALF = 16
_UNITS = _BATCH * _PW // _HALF
_NSLOT = 5
_GLEAD = 3
_LANES = 16


def _body(x_hbm, tok_hbm, pos_hbm, out_hbm, idx_v, pos_v, rows0, rows1,
          mini, *sems):
    base = [rows0, rows0, rows1, rows1, mini]
    roff = [0, _HALF, 0, _HALF, 0]
    gsem = sems[0:_NSLOT]
    osem = sems[_NSLOT:2 * _NSLOT]
    psem = sems[2 * _NSLOT]

    wid = lax.axis_index("s") * _NC + lax.axis_index("c")
    p0 = wid * _PW

    gd = {}
    od = {}

    def _addr(u):
        h = u // (2 * _BATCH)
        b = (u // 2) % _BATCH
        row = h * _SUB + (u % 2) * _HALF
        return h, b, row

    def gather(u):
        k = u % _NSLOT
        h, b, row = _addr(u)
        gd[u] = pltpu.async_copy(
            tok_hbm.at[idx_v.at[b, pl.ds(row, _HALF)]],
            base[k].at[pl.ds(roff[k], _HALF)], gsem[k])

    def outcopy(u):
        k = u % _NSLOT
        h, b, row = _addr(u)
        od[u] = pltpu.async_copy(
            base[k].at[pl.ds(roff[k], _HALF)],
            out_hbm.at[b, pl.ds(p0 + row, _HALF)], osem[k])

    def stage_pos(h):
        return pltpu.async_copy(
            pos_hbm.at[pl.ds(p0 + h * _SUB, _SUB)], pos_v, psem)

    def add_pos(u):
        k = u % _NSLOT
        buf = base[k]
        r0 = roff[k]
        p0r = (u % 2) * _HALF

        def add_row(r, _):
            for d in range(_DIM // _LANES):
                sl = pl.ds(d * _LANES, _LANES)
                plsc.addupdate(buf.at[r0 + r, sl], pos_v[p0r + r, sl])
            return 0

        lax.fori_loop(0, _HALF, add_row, 0)

    pltpu.sync_copy(x_hbm.at[0, pl.ds(p0, _PW)], idx_v.at[0])
    gather(0)
    gather(1)
    pltpu.sync_copy(x_hbm.at[1, pl.ds(p0, _PW)], idx_v.at[1])
    gather(2)
    pd = stage_pos(0)
    pltpu.sync_copy(x_hbm.at[2, pl.ds(p0, _PW)], idx_v.at[2])
    pltpu.sync_copy(x_hbm.at[3, pl.ds(p0, _PW)], idx_v.at[3])
    for u in range(_UNITS):
        ku = u + _GLEAD
        if ku < _UNITS:
            if ku >= _NSLOT:
                od[ku - _NSLOT].wait()
            gather(ku)
        gd[u].wait()
        if u == 0 or u == 2 * _BATCH:
            pd.wait()
        add_pos(u)
        outcopy(u)
        if u == 2 * _BATCH - 1:
            pd = stage_pos(1)
    for u in range(_UNITS - _NSLOT, _UNITS):
        od[u].wait()


def kernel(x, token_table, pos_table):
    mesh = plsc.VectorSubcoreMesh(core_axis_name="c", subcore_axis_name="s")
    run = functools.partial(
        pl.kernel,
        mesh=mesh,
        out_type=jax.ShapeDtypeStruct((_BATCH, _CTX, _DIM), jnp.float32),
        scratch_types=(
            [pltpu.VMEM((_BATCH, _PW), jnp.int32),
             pltpu.VMEM((_SUB, _DIM), jnp.float32),
             pltpu.VMEM((_SUB, _DIM), jnp.float32),
             pltpu.VMEM((_SUB, _DIM), jnp.float32),
             pltpu.VMEM((_HALF, _DIM), jnp.float32)]
            + [pltpu.SemaphoreType.DMA] * (2 * _NSLOT + 1)
        ),
    )(_body)
    return run(x, token_table, pos_table)


# speedup vs baseline: 1.0219x; 1.0219x over previous
import functools

import jax
import jax.numpy as jnp
from jax import lax
from jax.experimental import pallas as pl
from jax.experimental.pallas import tpu as pltpu
from jax.experimental.pallas import tpu_sc as plsc

_VOCAB = 100000
_CTX = 2048
_DIM = 1024
_BATCH = 4

_NC = 2
_NS = 16
_NW = _NC * _NS
_PW = _CTX // _NW
_SUB = 32
_HALF = _SUB // 2
_STEPS = 2 * _BATCH
_NBUF = 2
_LANES = 16


def _body(x_hbm, tok_hbm, pos_hbm, out_hbm, idx_v, pos_v, rows0, rows1,
          *sems):
    rows = [rows0, rows1]
    gsem = [sems[0:2], sems[2:4]]
    osem = [sems[4:6], sems[6:8]]
    psem = sems[8]

    wid = lax.axis_index("s") * _NC + lax.axis_index("c")
    p0 = wid * _PW

    gd = {}
    od = {}

    def gather(s, half):
        h, b = divmod(s, _BATCH)
        gd[s, half] = pltpu.async_copy(
            tok_hbm.at[idx_v.at[b, pl.ds(h * _SUB + half * _HALF, _HALF)]],
            rows[s % _NBUF].at[pl.ds(half * _HALF, _HALF)],
            gsem[s % _NBUF][half])

    def outcopy(s, half):
        h, b = divmod(s, _BATCH)
        od[s, half] = pltpu.async_copy(
            rows[s % _NBUF].at[pl.ds(half * _HALF, _HALF)],
            out_hbm.at[b, pl.ds(p0 + h * _SUB + half * _HALF, _HALF)],
            osem[s % _NBUF][half])

    def stage_pos(h):
        return pltpu.async_copy(
            pos_hbm.at[pl.ds(p0 + h * _SUB, _SUB)], pos_v, psem)

    def add_pos(s, half):
        buf = rows[s % _NBUF]

        def add_row(r, _):
            for d in range(_DIM // _LANES):
                sl = pl.ds(d * _LANES, _LANES)
                plsc.addupdate(buf.at[r, sl], pos_v[r, sl])
            return 0

        lax.fori_loop(half * _HALF, (half + 1) * _HALF, add_row, 0,
                      unroll=2)

    pltpu.sync_copy(x_hbm.at[0, pl.ds(p0, _PW)], idx_v.at[0])
    gather(0, 0)
    gather(0, 1)
    pltpu.sync_copy(x_hbm.at[1, pl.ds(p0, _PW)], idx_v.at[1])
    gather(1, 0)
    gather(1, 1)
    pd = stage_pos(0)
    pltpu.sync_copy(x_hbm.at[2, pl.ds(p0, _PW)], idx_v.at[2])
    pltpu.sync_copy(x_hbm.at[3, pl.ds(p0, _PW)], idx_v.at[3])
    for s in range(_STEPS):
        more = s + 1 < _STEPS
        if s >= 1 and more:
            od[s - 1, 0].wait()
            gather(s + 1, 0)
        gd[s, 0].wait()
        if s == 0 or s == _BATCH:
            pd.wait()
        add_pos(s, 0)
        outcopy(s, 0)
        if s >= 1 and more:
            od[s - 1, 1].wait()
            gather(s + 1, 1)
        gd[s, 1].wait()
        add_pos(s, 1)
        outcopy(s, 1)
        if s == _BATCH - 1:
            pd = stage_pos(1)
    for s in (_STEPS - 2, _STEPS - 1):
        od[s, 0].wait()
        od[s, 1].wait()


def kernel(x, token_table, pos_table):
    mesh = plsc.VectorSubcoreMesh(core_axis_name="c", subcore_axis_name="s")
    run = functools.partial(
        pl.kernel,
        mesh=mesh,
        out_type=jax.ShapeDtypeStruct((_BATCH, _CTX, _DIM), jnp.float32),
        scratch_types=(
            [pltpu.VMEM((_BATCH, _PW), jnp.int32),
             pltpu.VMEM((_SUB, _DIM), jnp.float32),
             pltpu.VMEM((_SUB, _DIM), jnp.float32),
             pltpu.VMEM((_SUB, _DIM), jnp.float32)]
            + [pltpu.SemaphoreType.DMA] * 9
        ),
    )(_body)
    return run(x, token_table, pos_table)


# speedup vs baseline: 1.1639x; 1.1390x over previous
import functools

import jax
import jax.numpy as jnp
from jax import lax
from jax.experimental import pallas as pl
from jax.experimental.pallas import tpu as pltpu
from jax.experimental.pallas import tpu_sc as plsc

_VOCAB = 100000
_CTX = 2048
_DIM = 1024
_BATCH = 4

_NC = 2
_NS = 16
_NW = _NC * _NS
_PW = _CTX // _NW
_SUB = 32
_HALF = _SUB // 2
_STEPS = 2 * _BATCH
_NBUF = 2
_LANES = 16


def _body(x_hbm, tok_hbm, pos_hbm, out_hbm, idx_v, pos_v, rows0, rows1,
          *sems):
    rows = [rows0, rows1]
    gsem = [sems[0:2], sems[2:4]]
    osem = [sems[4:6], sems[6:8]]
    psem = sems[8]

    wid = lax.axis_index("s") * _NC + lax.axis_index("c")
    p0 = wid * _PW

    gd = {}
    od = {}

    def gather(s, half):
        h, b = divmod(s, _BATCH)
        gd[s, half] = pltpu.async_copy(
            tok_hbm.at[idx_v.at[b, pl.ds(h * _SUB + half * _HALF, _HALF)]],
            rows[s % _NBUF].at[pl.ds(half * _HALF, _HALF)],
            gsem[s % _NBUF][half])

    def outcopy(s, half):
        h, b = divmod(s, _BATCH)
        od[s, half] = pltpu.async_copy(
            rows[s % _NBUF].at[pl.ds(half * _HALF, _HALF)],
            out_hbm.at[b, pl.ds(p0 + h * _SUB + half * _HALF, _HALF)],
            osem[s % _NBUF][half])

    def stage_pos(h):
        return pltpu.async_copy(
            pos_hbm.at[pl.ds(p0 + h * _SUB, _SUB)], pos_v, psem)

    def add_pos(s, half):
        buf = rows[s % _NBUF]

        def add_row(r, _):
            for d in range(_DIM // _LANES):
                sl = pl.ds(d * _LANES, _LANES)
                plsc.addupdate(buf.at[r, sl], pos_v[r, sl])
            return 0

        lax.fori_loop(half * _HALF, (half + 1) * _HALF, add_row, 0)

    pltpu.sync_copy(x_hbm.at[0, pl.ds(p0, _PW)], idx_v.at[0])
    gather(0, 0)
    gather(0, 1)
    pltpu.sync_copy(x_hbm.at[1, pl.ds(p0, _PW)], idx_v.at[1])
    gather(1, 0)
    gather(1, 1)
    pd = stage_pos(0)
    pltpu.sync_copy(x_hbm.at[2, pl.ds(p0, _PW)], idx_v.at[2])
    pltpu.sync_copy(x_hbm.at[3, pl.ds(p0, _PW)], idx_v.at[3])
    for s in range(_STEPS):
        more = s + 1 < _STEPS
        if s >= 1 and more:
            od[s - 1, 0].wait()
            gather(s + 1, 0)
        gd[s, 0].wait()
        if s == 0 or s == _BATCH:
            pd.wait()
        add_pos(s, 0)
        outcopy(s, 0)
        if s >= 1 and more:
            od[s - 1, 1].wait()
            gather(s + 1, 1)
        gd[s, 1].wait()
        add_pos(s, 1)
        outcopy(s, 1)
        if s == _BATCH - 1:
            pd = stage_pos(1)
    for s in (_STEPS - 2, _STEPS - 1):
        od[s, 0].wait()
        od[s, 1].wait()


def kernel(x, token_table, pos_table):
    mesh = plsc.VectorSubcoreMesh(core_axis_name="c", subcore_axis_name="s")
    run = functools.partial(
        pl.kernel,
        mesh=mesh,
        out_type=jax.ShapeDtypeStruct((_BATCH, _CTX, _DIM), jnp.float32),
        scratch_types=(
            [pltpu.VMEM((_BATCH, _PW), jnp.int32),
             pltpu.VMEM((_SUB, _DIM), jnp.float32),
             pltpu.VMEM((_SUB, _DIM), jnp.float32),
             pltpu.VMEM((_SUB, _DIM), jnp.float32)]
            + [pltpu.SemaphoreType.DMA] * 9
        ),
    )(_body)
    return run(x, token_table, pos_table)
